# initial kernel scaffold (unmeasured)
import jax
import jax.numpy as jnp
from jax import lax
from jax.experimental import pallas as pl
from jax.experimental.pallas import tpu as pltpu

T_SH = 256
D = 512
F = 1024
E_LOC = 2

NEG = jnp.float32(-1e30)


def kernel(x, router, W1, W2):
    def body(x_ref, r_ref, w1_ref, w2_ref, out_ref,
             xpeer_ref, rpeer_ref, pa_src, pa_dst, pb_src, pb_dst,
             send_sems, recv_sems):
        my_x = lax.axis_index("x")
        my_y = lax.axis_index("y")
        ypeer = (my_x, 1 - my_y)
        xpeer = (1 - my_x, my_y)

        barrier = pltpu.get_barrier_semaphore()
        for nbr in (ypeer, xpeer):
            pl.semaphore_signal(
                barrier, inc=1, device_id=nbr,
                device_id_type=pl.DeviceIdType.MESH,
            )
        pl.semaphore_wait(barrier, 2)

        rdma_x = pltpu.make_async_remote_copy(
            src_ref=x_ref, dst_ref=xpeer_ref,
            send_sem=send_sems.at[0], recv_sem=recv_sems.at[0],
            device_id=ypeer, device_id_type=pl.DeviceIdType.MESH,
        )
        rdma_r = pltpu.make_async_remote_copy(
            src_ref=r_ref, dst_ref=rpeer_ref,
            send_sem=send_sems.at[1], recv_sem=recv_sems.at[1],
            device_id=ypeer, device_id_type=pl.DeviceIdType.MESH,
        )
        rdma_x.start()
        rdma_r.start()
        rdma_x.wait()
        rdma_r.wait()

        tok = jnp.where(my_x == my_y, x_ref[...], xpeer_ref[...])

        g_loc = jnp.dot(tok, r_ref[...],
                        preferred_element_type=jnp.float32)
        g_peer = jnp.dot(tok, rpeer_ref[...],
                         preferred_element_type=jnp.float32)
        g = jnp.concatenate([g_loc, g_peer], axis=1)

        m1 = jnp.max(g, axis=1, keepdims=True)
        m2 = jnp.max(jnp.where(g == m1, NEG, g), axis=1, keepdims=True)
        eb = jnp.exp(m2 - m1)
        w_top1 = 1.0 / (1.0 + eb)
        w_top2 = eb / (1.0 + eb)

        partial = jnp.zeros((T_SH, D), jnp.float32)
        for s in range(E_LOC):
            ge = g_loc[:, s:s + 1]
            w = jnp.where(ge == m1, w_top1,
                          jnp.where(ge == m2, w_top2, 0.0))
            h = jnp.maximum(
                jnp.dot(tok, w1_ref[s], preferred_element_type=jnp.float32),
                0.0)
            y = jnp.dot(h, w2_ref[s], preferred_element_type=jnp.float32)
            partial = partial + y * w
        pa_src[...] = partial

        rdma_a = pltpu.make_async_remote_copy(
            src_ref=pa_src, dst_ref=pa_dst,
            send_sem=send_sems.at[2], recv_sem=recv_sems.at[2],
            device_id=ypeer, device_id_type=pl.DeviceIdType.MESH,
        )
        rdma_a.start()
        rdma_a.wait()
        pb_src[...] = pa_src[...] + pa_dst[...]

        rdma_b = pltpu.make_async_remote_copy(
            src_ref=pb_src, dst_ref=pb_dst,
            send_sem=send_sems.at[3], recv_sem=recv_sems.at[3],
            device_id=xpeer, device_id_type=pl.DeviceIdType.MESH,
        )
        rdma_b.start()
        rdma_b.wait()
        out_ref[...] = jnp.where(my_x == my_y, pb_src[...], pb_dst[...])

    return pl.pallas_call(
        body,
        out_shape=jax.ShapeDtypeStruct((T_SH, D), jnp.float32),
        in_specs=[pl.BlockSpec(memory_space=pltpu.VMEM)] * 4,
        out_specs=pl.BlockSpec(memory_space=pltpu.VMEM),
        scratch_shapes=[
            pltpu.VMEM((T_SH, D), jnp.float32),
            pltpu.VMEM((D, E_LOC), jnp.float32),
            pltpu.VMEM((T_SH, D), jnp.float32),
            pltpu.VMEM((T_SH, D), jnp.float32),
            pltpu.VMEM((T_SH, D), jnp.float32),
            pltpu.VMEM((T_SH, D), jnp.float32),
            pltpu.SemaphoreType.DMA((4,)),
            pltpu.SemaphoreType.DMA((4,)),
        ],
        compiler_params=pltpu.CompilerParams(collective_id=0),
    )(x, router, W1, W2)


# baseline (device time: 36838 ns/iter reference)
import jax
import jax.numpy as jnp
from jax import lax
from jax.experimental import pallas as pl
from jax.experimental.pallas import tpu as pltpu

T_SH = 256
D = 512
F = 1024
E_LOC = 2

NEG = -1e30


def kernel(x, router, W1, W2):
    def body(x_ref, r_ref, w1_ref, w2_ref, out_ref,
             xpeer_ref, rpeer_ref, pa_src, pa_dst, pb_src, pb_dst,
             send_sems, recv_sems):
        my_x = lax.axis_index("x")
        my_y = lax.axis_index("y")
        ypeer = (my_x, 1 - my_y)
        xpeer = (1 - my_x, my_y)

        barrier = pltpu.get_barrier_semaphore()
        for nbr in (ypeer, xpeer):
            pl.semaphore_signal(
                barrier, inc=1, device_id=nbr,
                device_id_type=pl.DeviceIdType.MESH,
            )
        pl.semaphore_wait(barrier, 2)

        rdma_x = pltpu.make_async_remote_copy(
            src_ref=x_ref, dst_ref=xpeer_ref,
            send_sem=send_sems.at[0], recv_sem=recv_sems.at[0],
            device_id=ypeer, device_id_type=pl.DeviceIdType.MESH,
        )
        rdma_r = pltpu.make_async_remote_copy(
            src_ref=r_ref, dst_ref=rpeer_ref,
            send_sem=send_sems.at[1], recv_sem=recv_sems.at[1],
            device_id=ypeer, device_id_type=pl.DeviceIdType.MESH,
        )
        rdma_x.start()
        rdma_r.start()
        rdma_x.wait()
        rdma_r.wait()

        tok = jnp.where(my_x == my_y, x_ref[...], xpeer_ref[...])

        g_loc = jnp.dot(tok, r_ref[...],
                        preferred_element_type=jnp.float32)
        g_peer = jnp.dot(tok, rpeer_ref[...],
                         preferred_element_type=jnp.float32)
        g = jnp.concatenate([g_loc, g_peer], axis=1)

        m1 = jnp.max(g, axis=1, keepdims=True)
        m2 = jnp.max(jnp.where(g == m1, NEG, g), axis=1, keepdims=True)
        eb = jnp.exp(m2 - m1)
        w_top1 = 1.0 / (1.0 + eb)
        w_top2 = eb / (1.0 + eb)

        partial = jnp.zeros((T_SH, D), jnp.float32)
        for s in range(E_LOC):
            ge = g_loc[:, s:s + 1]
            w = jnp.where(ge == m1, w_top1,
                          jnp.where(ge == m2, w_top2, 0.0))
            h = jnp.maximum(
                jnp.dot(tok, w1_ref[s], preferred_element_type=jnp.float32),
                0.0)
            y = jnp.dot(h, w2_ref[s], preferred_element_type=jnp.float32)
            partial = partial + y * w
        pa_src[...] = partial

        rdma_a = pltpu.make_async_remote_copy(
            src_ref=pa_src, dst_ref=pa_dst,
            send_sem=send_sems.at[2], recv_sem=recv_sems.at[2],
            device_id=ypeer, device_id_type=pl.DeviceIdType.MESH,
        )
        rdma_a.start()
        rdma_a.wait()
        pb_src[...] = pa_src[...] + pa_dst[...]

        rdma_b = pltpu.make_async_remote_copy(
            src_ref=pb_src, dst_ref=pb_dst,
            send_sem=send_sems.at[3], recv_sem=recv_sems.at[3],
            device_id=xpeer, device_id_type=pl.DeviceIdType.MESH,
        )
        rdma_b.start()
        rdma_b.wait()
        out_ref[...] = jnp.where(my_x == my_y, pb_src[...], pb_dst[...])

    return pl.pallas_call(
        body,
        out_shape=jax.ShapeDtypeStruct((T_SH, D), jnp.float32),
        in_specs=[pl.BlockSpec(memory_space=pltpu.VMEM)] * 4,
        out_specs=pl.BlockSpec(memory_space=pltpu.VMEM),
        scratch_shapes=[
            pltpu.VMEM((T_SH, D), jnp.float32),
            pltpu.VMEM((D, E_LOC), jnp.float32),
            pltpu.VMEM((T_SH, D), jnp.float32),
            pltpu.VMEM((T_SH, D), jnp.float32),
            pltpu.VMEM((T_SH, D), jnp.float32),
            pltpu.VMEM((T_SH, D), jnp.float32),
            pltpu.SemaphoreType.DMA((4,)),
            pltpu.SemaphoreType.DMA((4,)),
        ],
        compiler_params=pltpu.CompilerParams(collective_id=0),
    )(x, router, W1, W2)


# device time: 29425 ns/iter; 1.2519x vs baseline; 1.2519x over previous
import jax
import jax.numpy as jnp
from jax import lax
from jax.experimental import pallas as pl
from jax.experimental.pallas import tpu as pltpu

T_SH = 256
D = 512
F = 1024
E_LOC = 2
NC = 4
C = T_SH // NC

NEG = -1e30


def kernel(x, router, W1, W2):
    def body(x_ref, r_ref, w1_ref, w2_ref, out_ref,
             xpeer_ref, rpeer_ref, w1b_ref, w2b_ref,
             pa_src, pa_dst, pb_src, pb_dst,
             sx_s, sx_r, sr_s, sr_r, sa_s, sa_r, sb_s, sb_r):
        my_x = lax.axis_index("x")
        my_y = lax.axis_index("y")
        ypeer = (my_x, 1 - my_y)
        xpeer = (1 - my_x, my_y)

        barrier = pltpu.get_barrier_semaphore()
        for nbr in (ypeer, xpeer):
            pl.semaphore_signal(
                barrier, inc=1, device_id=nbr,
                device_id_type=pl.DeviceIdType.MESH,
            )
        pl.semaphore_wait(barrier, 2)

        rdma_r = pltpu.make_async_remote_copy(
            src_ref=r_ref, dst_ref=rpeer_ref,
            send_sem=sr_s.at[0], recv_sem=sr_r.at[0],
            device_id=ypeer, device_id_type=pl.DeviceIdType.MESH,
        )
        rdma_r.start()
        rdma_x = []
        for c in range(NC):
            r = pltpu.make_async_remote_copy(
                src_ref=x_ref.at[pl.ds(c * C, C)],
                dst_ref=xpeer_ref.at[pl.ds(c * C, C)],
                send_sem=sx_s.at[c], recv_sem=sx_r.at[c],
                device_id=ypeer, device_id_type=pl.DeviceIdType.MESH,
            )
            r.start()
            rdma_x.append(r)

        w1b_ref[...] = w1_ref[...].astype(jnp.bfloat16)
        w2b_ref[...] = w2_ref[...].astype(jnp.bfloat16)
        rdma_r.wait()

        def reduce_and_swap(c, rdma_a_c):
            rdma_a_c.wait()
            sl = pl.ds(c * C, C)
            pb_src[sl, :] = pa_src[sl, :] + pa_dst[sl, :]
            r = pltpu.make_async_remote_copy(
                src_ref=pb_src.at[sl], dst_ref=pb_dst.at[sl],
                send_sem=sb_s.at[c], recv_sem=sb_r.at[c],
                device_id=xpeer, device_id_type=pl.DeviceIdType.MESH,
            )
            r.start()
            return r

        rdma_a = []
        rdma_b = []
        for c in range(NC):
            rdma_x[c].wait()
            sl = pl.ds(c * C, C)
            tok = jnp.where(my_x == my_y, x_ref[sl, :], xpeer_ref[sl, :])

            g_loc = jnp.dot(tok, r_ref[...],
                            preferred_element_type=jnp.float32)
            g_peer = jnp.dot(tok, rpeer_ref[...],
                             preferred_element_type=jnp.float32)
            g = jnp.concatenate([g_loc, g_peer], axis=1)
            m1 = jnp.max(g, axis=1, keepdims=True)
            m2 = jnp.max(jnp.where(g == m1, NEG, g), axis=1, keepdims=True)
            eb = jnp.exp(m2 - m1)
            w_top1 = 1.0 / (1.0 + eb)
            w_top2 = eb / (1.0 + eb)

            tokb = tok.astype(jnp.bfloat16)
            partial = jnp.zeros((C, D), jnp.float32)
            for s in range(E_LOC):
                ge = g_loc[:, s:s + 1]
                w = jnp.where(ge == m1, w_top1,
                              jnp.where(ge == m2, w_top2, 0.0))
                h = jnp.maximum(
                    jnp.dot(tokb, w1b_ref[s],
                            preferred_element_type=jnp.float32), 0.0)
                y = jnp.dot(h.astype(jnp.bfloat16), w2b_ref[s],
                            preferred_element_type=jnp.float32)
                partial = partial + y * w
            pa_src[sl, :] = partial

            r = pltpu.make_async_remote_copy(
                src_ref=pa_src.at[sl], dst_ref=pa_dst.at[sl],
                send_sem=sa_s.at[c], recv_sem=sa_r.at[c],
                device_id=ypeer, device_id_type=pl.DeviceIdType.MESH,
            )
            r.start()
            rdma_a.append(r)

            if c >= 1:
                rdma_b.append(reduce_and_swap(c - 1, rdma_a[c - 1]))

        rdma_b.append(reduce_and_swap(NC - 1, rdma_a[NC - 1]))
        for c in range(NC):
            rdma_b[c].wait()
        out_ref[...] = jnp.where(my_x == my_y, pb_src[...], pb_dst[...])

    return pl.pallas_call(
        body,
        out_shape=jax.ShapeDtypeStruct((T_SH, D), jnp.float32),
        in_specs=[pl.BlockSpec(memory_space=pltpu.VMEM)] * 4,
        out_specs=pl.BlockSpec(memory_space=pltpu.VMEM),
        scratch_shapes=[
            pltpu.VMEM((T_SH, D), jnp.float32),
            pltpu.VMEM((D, E_LOC), jnp.float32),
            pltpu.VMEM((E_LOC, D, F), jnp.bfloat16),
            pltpu.VMEM((E_LOC, F, D), jnp.bfloat16),
            pltpu.VMEM((T_SH, D), jnp.float32),
            pltpu.VMEM((T_SH, D), jnp.float32),
            pltpu.VMEM((T_SH, D), jnp.float32),
            pltpu.VMEM((T_SH, D), jnp.float32),
            pltpu.SemaphoreType.DMA((NC,)),
            pltpu.SemaphoreType.DMA((NC,)),
            pltpu.SemaphoreType.DMA((1,)),
            pltpu.SemaphoreType.DMA((1,)),
            pltpu.SemaphoreType.DMA((NC,)),
            pltpu.SemaphoreType.DMA((NC,)),
            pltpu.SemaphoreType.DMA((NC,)),
            pltpu.SemaphoreType.DMA((NC,)),
        ],
        compiler_params=pltpu.CompilerParams(collective_id=0),
    )(x, router, W1, W2)


# device time: 22851 ns/iter; 1.6121x vs baseline; 1.2877x over previous
import jax
import jax.numpy as jnp
from jax import lax
from jax.experimental import pallas as pl
from jax.experimental.pallas import tpu as pltpu

T_SH = 256
D = 512
F = 1024
E_LOC = 2
NC = 4
C = T_SH // NC

NEG = -1e30
MESH = pl.DeviceIdType.MESH


def kernel(x, router, W1, W2):
    def body(x_ref, r_ref, w1_ref, w2_ref, out_ref,
             xb_src, xb_dst, rpeer_ref, pa_src, pa_dst, pb_src, pb_dst,
             sx_s, sx_r, sr_s, sr_r, sa_s, sa_r, sb_s, sb_r, credit):
        my_x = lax.axis_index("x")
        my_y = lax.axis_index("y")
        ypeer = (my_x, 1 - my_y)
        xpeer = (1 - my_x, my_y)
        is_owner = my_x == my_y

        barrier = pltpu.get_barrier_semaphore()
        for nbr in (ypeer, xpeer):
            pl.semaphore_signal(barrier, inc=1, device_id=nbr,
                                device_id_type=MESH)
        pl.semaphore_wait(barrier, 2)

        @pl.when(jnp.logical_not(is_owner))
        def _():
            pl.semaphore_signal(credit, inc=1, device_id=xpeer,
                                device_id_type=MESH)

        def x_rdma(c):
            sl = pl.ds(c * C, C)
            return pltpu.make_async_remote_copy(
                src_ref=xb_src.at[sl], dst_ref=xb_dst.at[sl],
                send_sem=sx_s.at[c], recv_sem=sx_r.at[c],
                device_id=ypeer, device_id_type=MESH)

        def a_rdma(c):
            sl = pl.ds(c * C, C)
            return pltpu.make_async_remote_copy(
                src_ref=pa_src.at[sl], dst_ref=pa_dst.at[sl],
                send_sem=sa_s.at[c], recv_sem=sa_r.at[c],
                device_id=ypeer, device_id_type=MESH)

        def b_rdma(c):
            sl = pl.ds(c * C, C)
            return pltpu.make_async_remote_copy(
                src_ref=pb_src.at[sl], dst_ref=pb_dst.at[sl],
                send_sem=sb_s.at[c], recv_sem=sb_r.at[c],
                device_id=xpeer, device_id_type=MESH)

        rdma_r = pltpu.make_async_remote_copy(
            src_ref=r_ref, dst_ref=rpeer_ref,
            send_sem=sr_s.at[0], recv_sem=sr_r.at[0],
            device_id=ypeer, device_id_type=MESH)
        rdma_r.start()
        xb_src[...] = x_ref[...].astype(jnp.bfloat16)

        @pl.when(is_owner)
        def _():
            for c in range(NC):
                x_rdma(c).start()

        rdma_r.wait()

        def finish_chunk(c):
            a_rdma(c).wait_recv()
            sl = pl.ds(c * C, C)
            s = (pa_src[sl, :].astype(jnp.float32)
                 + pa_dst[sl, :].astype(jnp.float32))
            out_ref[sl, :] = s
            pb_src[sl, :] = s.astype(jnp.bfloat16)
            if c == 0:
                pl.semaphore_wait(credit, 1)
            b_rdma(c).start()

        for c in range(NC):
            sl = pl.ds(c * C, C)

            @pl.when(jnp.logical_not(is_owner))
            def _():
                x_rdma(c).wait_recv()

            tokb = jnp.where(is_owner, xb_src[sl, :], xb_dst[sl, :])
            tok = tokb.astype(jnp.float32)

            g_loc = jnp.dot(tok, r_ref[...],
                            preferred_element_type=jnp.float32)
            g_peer = jnp.dot(tok, rpeer_ref[...],
                             preferred_element_type=jnp.float32)
            g = jnp.concatenate([g_loc, g_peer], axis=1)
            m1 = jnp.max(g, axis=1, keepdims=True)
            m2 = jnp.max(jnp.where(g == m1, NEG, g), axis=1, keepdims=True)
            eb = jnp.exp(m2 - m1)
            w_top1 = 1.0 / (1.0 + eb)
            w_top2 = eb / (1.0 + eb)

            partial = jnp.zeros((C, D), jnp.float32)
            for s in range(E_LOC):
                ge = g_loc[:, s:s + 1]
                w = jnp.where(ge == m1, w_top1,
                              jnp.where(ge == m2, w_top2, 0.0))
                h = jnp.maximum(
                    jnp.dot(tok, w1_ref[s],
                            preferred_element_type=jnp.float32), 0.0)
                y = jnp.dot(h, w2_ref[s],
                            preferred_element_type=jnp.float32)
                partial = partial + y * w
            pa_src[sl, :] = partial.astype(jnp.bfloat16)

            @pl.when(jnp.logical_not(is_owner))
            def _():
                a_rdma(c).start()

            if c >= 1:
                @pl.when(is_owner)
                def _():
                    finish_chunk(c - 1)

        @pl.when(is_owner)
        def _():
            finish_chunk(NC - 1)
            for c in range(NC):
                b_rdma(c).wait_send()
                x_rdma(c).wait_send()

        @pl.when(jnp.logical_not(is_owner))
        def _():
            for c in range(NC):
                b_rdma(c).wait_recv()
                a_rdma(c).wait_send()
            out_ref[...] = pb_dst[...].astype(jnp.float32)

    return pl.pallas_call(
        body,
        out_shape=jax.ShapeDtypeStruct((T_SH, D), jnp.float32),
        in_specs=[pl.BlockSpec(memory_space=pltpu.VMEM)] * 4,
        out_specs=pl.BlockSpec(memory_space=pltpu.VMEM),
        scratch_shapes=[
            pltpu.VMEM((T_SH, D), jnp.bfloat16),
            pltpu.VMEM((T_SH, D), jnp.bfloat16),
            pltpu.VMEM((D, E_LOC), jnp.float32),
            pltpu.VMEM((T_SH, D), jnp.bfloat16),
            pltpu.VMEM((T_SH, D), jnp.bfloat16),
            pltpu.VMEM((T_SH, D), jnp.bfloat16),
            pltpu.VMEM((T_SH, D), jnp.bfloat16),
            pltpu.SemaphoreType.DMA((NC,)),
            pltpu.SemaphoreType.DMA((NC,)),
            pltpu.SemaphoreType.DMA((1,)),
            pltpu.SemaphoreType.DMA((1,)),
            pltpu.SemaphoreType.DMA((NC,)),
            pltpu.SemaphoreType.DMA((NC,)),
            pltpu.SemaphoreType.DMA((NC,)),
            pltpu.SemaphoreType.DMA((NC,)),
            pltpu.SemaphoreType.REGULAR,
        ],
        compiler_params=pltpu.CompilerParams(collective_id=0),
    )(x, router, W1, W2)


# device time: 21351 ns/iter; 1.7254x vs baseline; 1.0703x over previous
import jax
import jax.numpy as jnp
from jax import lax
from jax.experimental import pallas as pl
from jax.experimental.pallas import tpu as pltpu

T_SH = 256
D = 512
F = 1024
E_LOC = 2
NC = 2
C = T_SH // NC

NEG = -1e30
MESH = pl.DeviceIdType.MESH


def kernel(x, router, W1, W2):
    def body(x_ref, r_ref, w1_ref, w2_ref, out_ref,
             xb_src, xb_dst, rpeer_ref, pa_src, pa_dst, px_dst, pd_dst,
             sx_s, sx_r, sr_s, sr_r, sa_s, sa_r, spx_s, spx_r,
             sd_s, sd_r, credit_b, credit_d, alloc2):
        my_x = lax.axis_index("x")
        my_y = lax.axis_index("y")
        ypeer = (my_x, 1 - my_y)
        xpeer = (1 - my_x, my_y)
        dpeer = (1 - my_x, 1 - my_y)
        is_owner = my_x == my_y
        not_owner = jnp.logical_not(is_owner)

        barrier = pltpu.get_barrier_semaphore()
        pl.semaphore_signal(barrier, inc=1, device_id=ypeer,
                            device_id_type=MESH)
        for nbr in (xpeer, dpeer):
            pl.semaphore_signal(alloc2, inc=1, device_id=nbr,
                                device_id_type=MESH)
        pl.semaphore_wait(barrier, 1)

        @pl.when(not_owner)
        def _():
            pl.semaphore_signal(credit_b, inc=1, device_id=xpeer,
                                device_id_type=MESH)
            pl.semaphore_signal(credit_d, inc=1, device_id=dpeer,
                                device_id_type=MESH)

        def mk(src, dst, ssem, rsem, c, dev):
            sl = pl.ds(c * C, C)
            return pltpu.make_async_remote_copy(
                src_ref=src.at[sl], dst_ref=dst.at[sl],
                send_sem=ssem.at[c], recv_sem=rsem.at[c],
                device_id=dev, device_id_type=MESH)

        def x_rdma(c):
            return mk(xb_src, xb_dst, sx_s, sx_r, c, ypeer)

        def a_rdma(c):
            return mk(pa_src, pa_dst, sa_s, sa_r, c, ypeer)

        def px_rdma(c):
            return mk(pa_src, px_dst, spx_s, spx_r, c, xpeer)

        def d_rdma(c):
            return mk(pa_src, pd_dst, sd_s, sd_r, c, dpeer)

        rdma_r = pltpu.make_async_remote_copy(
            src_ref=r_ref, dst_ref=rpeer_ref,
            send_sem=sr_s.at[0], recv_sem=sr_r.at[0],
            device_id=ypeer, device_id_type=MESH)
        rdma_r.start()
        xb_src[...] = x_ref[...].astype(jnp.bfloat16)

        @pl.when(is_owner)
        def _():
            for c in range(NC):
                x_rdma(c).start()

        rdma_r.wait()

        for c in range(NC):
            sl = pl.ds(c * C, C)

            @pl.when(not_owner)
            def _():
                x_rdma(c).wait_recv()

            tok = jnp.where(is_owner, xb_src[sl, :],
                            xb_dst[sl, :]).astype(jnp.float32)

            g_loc = jnp.dot(tok, r_ref[...],
                            preferred_element_type=jnp.float32)
            g_peer = jnp.dot(tok, rpeer_ref[...],
                             preferred_element_type=jnp.float32)
            g = jnp.concatenate([g_loc, g_peer], axis=1)
            m1 = jnp.max(g, axis=1, keepdims=True)
            m2 = jnp.max(jnp.where(g == m1, NEG, g), axis=1, keepdims=True)
            eb = jnp.exp(m2 - m1)
            w_top1 = 1.0 / (1.0 + eb)
            w_top2 = eb / (1.0 + eb)

            partial = jnp.zeros((C, D), jnp.float32)
            for s in range(E_LOC):
                ge = g_loc[:, s:s + 1]
                w = jnp.where(ge == m1, w_top1,
                              jnp.where(ge == m2, w_top2, 0.0))
                h = jnp.maximum(
                    jnp.dot(tok, w1_ref[s],
                            preferred_element_type=jnp.float32), 0.0)
                y = jnp.dot(h, w2_ref[s],
                            preferred_element_type=jnp.float32)
                partial = partial + y * w
            pa_src[sl, :] = partial.astype(jnp.bfloat16)

            @pl.when(not_owner)
            def _():
                if c == 0:
                    pl.semaphore_wait(alloc2, 2)
                    pl.semaphore_wait(credit_d, 1)
                a_rdma(c).start()
                d_rdma(c).start()

            @pl.when(is_owner)
            def _():
                if c == 0:
                    pl.semaphore_wait(alloc2, 2)
                    pl.semaphore_wait(credit_b, 1)
                px_rdma(c).start()

        @pl.when(is_owner)
        def _():
            for c in range(NC):
                a_rdma(c).wait_recv()
                sl = pl.ds(c * C, C)
                out_ref[sl, :] = (pa_src[sl, :].astype(jnp.float32)
                                  + pa_dst[sl, :].astype(jnp.float32))
            for c in range(NC):
                x_rdma(c).wait_send()
                px_rdma(c).wait_send()

        @pl.when(not_owner)
        def _():
            for c in range(NC):
                px_rdma(c).wait_recv()
                d_rdma(c).wait_recv()
                sl = pl.ds(c * C, C)
                out_ref[sl, :] = (px_dst[sl, :].astype(jnp.float32)
                                  + pd_dst[sl, :].astype(jnp.float32))
            for c in range(NC):
                a_rdma(c).wait_send()
                d_rdma(c).wait_send()

    return pl.pallas_call(
        body,
        out_shape=jax.ShapeDtypeStruct((T_SH, D), jnp.float32),
        in_specs=[pl.BlockSpec(memory_space=pltpu.VMEM)] * 4,
        out_specs=pl.BlockSpec(memory_space=pltpu.VMEM),
        scratch_shapes=[
            pltpu.VMEM((T_SH, D), jnp.bfloat16),
            pltpu.VMEM((T_SH, D), jnp.bfloat16),
            pltpu.VMEM((D, E_LOC), jnp.float32),
            pltpu.VMEM((T_SH, D), jnp.bfloat16),
            pltpu.VMEM((T_SH, D), jnp.bfloat16),
            pltpu.VMEM((T_SH, D), jnp.bfloat16),
            pltpu.VMEM((T_SH, D), jnp.bfloat16),
            pltpu.SemaphoreType.DMA((NC,)),
            pltpu.SemaphoreType.DMA((NC,)),
            pltpu.SemaphoreType.DMA((1,)),
            pltpu.SemaphoreType.DMA((1,)),
            pltpu.SemaphoreType.DMA((NC,)),
            pltpu.SemaphoreType.DMA((NC,)),
            pltpu.SemaphoreType.DMA((NC,)),
            pltpu.SemaphoreType.DMA((NC,)),
            pltpu.SemaphoreType.DMA((NC,)),
            pltpu.SemaphoreType.DMA((NC,)),
            pltpu.SemaphoreType.REGULAR,
            pltpu.SemaphoreType.REGULAR,
            pltpu.SemaphoreType.REGULAR,
        ],
        compiler_params=pltpu.CompilerParams(collective_id=0),
    )(x, router, W1, W2)
